# baseline (speedup 1.0000x reference)
import jax
import jax.numpy as jnp
from jax import lax
from jax.experimental import pallas as pl
from jax.experimental.pallas import tpu as pltpu
from jax.experimental.pallas import tpu_sc as plsc

N = 1000000
C = 32
NUM_OUT = 262144
NUM_RANGES = 8
S = NUM_OUT // NUM_RANGES
PASSES = NUM_RANGES // 2
CHUNK = 2000
NCHUNK = N // CHUNK
VPC = CHUNK // 16
B = 64
OB = 128
ZB = 64
SPAN = S // 16
STAGE = CHUNK + 2 * B


def _body(feat_hbm, coords_hbm, out_hbm,
          sums_sh, cnts_sh,
          coords_v, ids_v, rel_v, idx_rows, rows_v, quad_v,
          ones_v, sums_o, cnts_o, out_stage):
    c = lax.axis_index("c")
    s = lax.axis_index("s")
    iota = lax.iota(jnp.int32, 16)
    f_one = jnp.full((16,), 1.0, jnp.float32)
    f_zero = jnp.full((16,), 0.0, jnp.float32)
    i_zero = jnp.full((16,), 0, jnp.int32)
    i_trash = jnp.full((16,), S, jnp.int32)

    def init_ones(i, _):
        ones_v[i] = f_one
        return 0
    lax.fori_loop(0, B, init_ones, 0)

    n_my = jnp.where(s < NCHUNK - 16 * (NCHUNK // 16), NCHUNK // 16 + 1,
                     NCHUNK // 16)

    for p in range(PASSES):
        rng = c * PASSES + p
        base = rng * S

        def zrow(i, _):
            sums_o[i, pl.ds(0, 16)] = f_zero
            sums_o[i, pl.ds(16, 16)] = f_zero
            cnts_o[i] = f_zero
            return 0
        lax.fori_loop(0, OB, zrow, 0)
        for blk in range(SPAN // OB):
            r0 = s * SPAN + blk * OB
            pltpu.sync_copy(sums_o, sums_sh.at[pl.ds(r0, OB)])
            pltpu.sync_copy(cnts_o, cnts_sh.at[pl.ds(r0, OB)])
        plsc.subcore_barrier()

        def do_chunk(k, _):
            chunk_id = s + k * 16
            point0 = chunk_id * CHUNK
            pltpu.sync_copy(coords_hbm.at[pl.ds(point0 * 3, CHUNK * 3)],
                            coords_v)

            def scan16(i, off):
                i3 = i * 48 + iota * 3
                x = plsc.load_gather(coords_v, [i3])
                y = plsc.load_gather(coords_v, [i3 + 1])
                z = plsc.load_gather(coords_v, [i3 + 2])
                seg = ((x >> 1) << 12) | ((y >> 1) << 6) | (z >> 1)
                m = (seg >> 15) == rng
                rel = seg & (S - 1)
                pid = point0 + i * 16 + iota
                plsc.store_compressed(ids_v.at[pl.ds(off, 16)], pid, mask=m)
                plsc.store_compressed(rel_v.at[pl.ds(off, 16)], rel, mask=m)
                return off + jnp.sum(m.astype(jnp.int32))

            m_cnt = lax.fori_loop(0, VPC, scan16, 0)

            for j in range(B // 16):
                ids_v[pl.ds(m_cnt + j * 16, 16)] = i_zero
                rel_v[pl.ds(m_cnt + j * 16, 16)] = i_trash

            def do_batch(j, _):
                o = j * B
                for t in range(B // 16):
                    pid_v = ids_v[pl.ds(o + t * 16, 16)]
                    idx_rows[0, pl.ds(t * 16, 16)] = rel_v[pl.ds(o + t * 16,
                                                                 16)]
                    idx_rows[1, pl.ds(t * 16, 16)] = pid_v >> 2
                pltpu.sync_copy(feat_hbm.at[idx_rows.at[1]], quad_v)
                for t in range(B // 16):
                    pid_v = ids_v[pl.ds(o + t * 16, 16)]
                    row_i = t * 16 + iota
                    colb = (pid_v & 3) * 32
                    for ch in range(C):
                        v = plsc.load_gather(quad_v, [row_i, colb + ch])
                        plsc.store_scatter(rows_v, [row_i, i_zero + ch], v)
                pltpu.sync_copy(rows_v,
                                sums_sh.at[idx_rows.at[0]], add=True)
                pltpu.sync_copy(ones_v, cnts_sh.at[idx_rows.at[0]], add=True)
                return 0

            lax.fori_loop(0, (m_cnt + B - 1) // B, do_batch, 0)
            return 0

        lax.fori_loop(0, n_my, do_chunk, 0)
        plsc.subcore_barrier()

        for blk in range(SPAN // OB):
            r0 = s * SPAN + blk * OB
            pltpu.sync_copy(sums_sh.at[pl.ds(r0, OB)], sums_o)
            pltpu.sync_copy(cnts_sh.at[pl.ds(r0, OB)], cnts_o)

            def divrow(rr, _):
                cm = jnp.maximum(cnts_o[rr], 1.0)
                orow = rr >> 2
                ocol = (rr & 3) * 32
                out_stage[orow, pl.ds(ocol, 16)] = \
                    sums_o[rr, pl.ds(0, 16)] / cm
                out_stage[orow, pl.ds(ocol + 16, 16)] = \
                    sums_o[rr, pl.ds(16, 16)] / cm
                return 0

            lax.fori_loop(0, OB, divrow, 0)
            pltpu.sync_copy(out_stage,
                            out_hbm.at[pl.ds((base + r0) // 4, OB // 4)])
        plsc.subcore_barrier()


@jax.jit
def _pooled(features, coords):
    mesh = plsc.VectorSubcoreMesh(core_axis_name="c", subcore_axis_name="s")
    f = pl.kernel(
        _body,
        out_type=jax.ShapeDtypeStruct((NUM_OUT * C // 128, 128), jnp.float32),
        mesh=mesh,
        compiler_params=pltpu.CompilerParams(needs_layout_passes=False,
                                             use_tc_tiling_on_sc=False),
        scratch_types=[
            pltpu.VMEM_SHARED((S + 1, C), jnp.float32),
            pltpu.VMEM_SHARED((S + 1, 16), jnp.float32),
            pltpu.VMEM((CHUNK * 3,), jnp.int32),
            pltpu.VMEM((STAGE,), jnp.int32),
            pltpu.VMEM((STAGE,), jnp.int32),
            pltpu.VMEM((2, B), jnp.int32),
            pltpu.VMEM((B, C), jnp.float32),
            pltpu.VMEM((B, 128), jnp.float32),
            pltpu.VMEM((B, 16), jnp.float32),
            pltpu.VMEM((OB, C), jnp.float32),
            pltpu.VMEM((OB, 16), jnp.float32),
            pltpu.VMEM((OB // 4, 128), jnp.float32),
        ],
    )
    return f(features, coords)


def kernel(features, coords):
    out = _pooled(features.reshape(N * C // 128, 128), coords.reshape(-1))
    return out.reshape(NUM_OUT, C)


# speedup vs baseline: 1.4475x; 1.4475x over previous
import jax
import jax.numpy as jnp
from jax import lax
from jax.experimental import pallas as pl
from jax.experimental.pallas import tpu as pltpu
from jax.experimental.pallas import tpu_sc as plsc

N = 1000000
C = 32
NUM_OUT = 262144
NUM_RANGES = 8
S = NUM_OUT // NUM_RANGES
PASSES = NUM_RANGES // 2
BLK = 256
NBLK = (N + BLK - 1) // BLK
GPB = BLK // 16
CW = BLK * 3
FW = BLK * C
RING = 512
FB = 128
OB = 64
SPAN = S // 16


def _body(feat_hbm, coords_hbm, out_hbm,
          sums_sh, cnts_sh,
          cbuf, fbuf, srcr_v, brel_v, ring_rel, cr_v,
          idx_rows, ones_st, sums_o, cnts_o, out_stage,
          csems, fsems):
    c = lax.axis_index("c")
    s = lax.axis_index("s")
    iota = lax.iota(jnp.int32, 16)
    f_one = jnp.full((16,), 1.0, jnp.float32)
    f_zero = jnp.full((16,), 0.0, jnp.float32)
    i_zero = jnp.full((16,), 0, jnp.int32)
    i_trash = jnp.full((16,), S, jnp.int32)

    def init_z(i, _):
        ones_st[i] = f_zero
        return 0
    lax.fori_loop(0, FB, init_z, 0)

    n_my = jnp.where(s < NBLK - 16 * (NBLK // 16), NBLK // 16 + 1,
                     NBLK // 16)

    def start_block(k, slot):
        b = s + k * 16
        pltpu.async_copy(coords_hbm.at[pl.ds(b * CW, CW)],
                         cbuf.at[pl.ds(slot * CW, CW)], csems.at[slot])
        pltpu.async_copy(feat_hbm.at[pl.ds(b * FW, FW)],
                         fbuf.at[pl.ds(slot * FW, FW)], fsems.at[slot])

    def wait_block(k, slot):
        b = s + k * 16
        pltpu.make_async_copy(coords_hbm.at[pl.ds(b * CW, CW)],
                              cbuf.at[pl.ds(slot * CW, CW)],
                              csems.at[slot]).wait()
        pltpu.make_async_copy(feat_hbm.at[pl.ds(b * FW, FW)],
                              fbuf.at[pl.ds(slot * FW, FW)],
                              fsems.at[slot]).wait()

    for p in range(PASSES):
        rng = c * PASSES + p
        base = rng * S

        def zrow(i, _):
            sums_o[i, pl.ds(0, 16)] = f_zero
            sums_o[i, pl.ds(16, 16)] = f_zero
            return 0
        lax.fori_loop(0, OB, zrow, 0)

        def zcrow(i, _):
            cnts_o[i] = f_zero
            return 0
        lax.fori_loop(0, OB // 4, zcrow, 0)

        def zblk(blk, _):
            r0 = pl.multiple_of(s * SPAN + blk * OB, OB)
            pltpu.sync_copy(sums_o, sums_sh.at[pl.ds(r0, OB)])
            pltpu.sync_copy(cnts_o, cnts_sh.at[pl.ds(r0 // 4, OB // 4)])
            return 0
        lax.fori_loop(0, SPAN // OB, zblk, 0)
        plsc.subcore_barrier()

        def fire_one(i, f0):
            h0 = pl.multiple_of((f0 + i * FB) & (RING - 1), FB)
            for t in range(FB // 16):
                rl = ring_rel[pl.ds(h0 + t * 16, 16)]
                idx_rows[0, pl.ds(t * 16, 16)] = rl
                idx_rows[1, pl.ds(t * 16, 16)] = rl >> 2
                plsc.store_scatter(ones_st, [t * 16 + iota, (rl & 3) * 4],
                                   f_one)
            pltpu.sync_copy(cr_v.at[pl.ds(h0, FB)],
                            sums_sh.at[idx_rows.at[0]], add=True)
            pltpu.sync_copy(ones_st, cnts_sh.at[idx_rows.at[1]], add=True)
            for t in range(FB // 16):
                rl = idx_rows[0, pl.ds(t * 16, 16)]
                plsc.store_scatter(ones_st, [t * 16 + iota, (rl & 3) * 4],
                                   f_zero)
            return f0

        def do_block(k, carry):
            off, fired = carry
            slot = k & 1

            @pl.when(k + 1 < n_my)
            def _():
                start_block(k + 1, slot ^ 1)

            wait_block(k, slot)
            c0 = slot * CW
            f0w = slot * FW

            def scan16(g, bs):
                w = c0 + g * 48 + iota * 3
                x = plsc.load_gather(cbuf, [w])
                y = plsc.load_gather(cbuf, [w + 1])
                z = plsc.load_gather(cbuf, [w + 2])
                seg = ((x >> 1) << 12) | ((y >> 1) << 6) | (z >> 1)
                m = (seg >> 15) == rng
                rel = seg & (S - 1)
                plsc.store_compressed(srcr_v.at[pl.ds(bs, 16)], g * 16 + iota,
                                      mask=m)
                plsc.store_compressed(brel_v.at[pl.ds(bs, 16)], rel, mask=m)
                return bs + jnp.sum(m.astype(jnp.int32))

            bs = lax.fori_loop(0, GPB, scan16, 0)

            srcr_v[pl.ds(bs, 16)] = i_zero
            brel_v[pl.ds(bs, 16)] = i_trash

            def compact(gi, _):
                sr = srcr_v[pl.ds(gi * 16, 16)]
                rl = brel_v[pl.ds(gi * 16, 16)]
                pos = (off + gi * 16 + iota) & (RING - 1)
                plsc.store_scatter(ring_rel, [pos], rl)

                def xch(ch, _):
                    v = plsc.load_gather(fbuf, [f0w + sr * C + ch])
                    plsc.store_scatter(cr_v, [pos, i_zero + ch], v)
                    return 0
                lax.fori_loop(0, C, xch, 0)
                return 0

            lax.fori_loop(0, (bs + 15) // 16, compact, 0)
            off = off + bs

            nf = (off - fired) // FB
            lax.fori_loop(0, nf, fire_one, fired)
            return (off, fired + nf * FB)

        start_block(0, 0)
        off, fired = lax.fori_loop(0, n_my, do_block, (0, 0))

        def padrest(j, _):
            pos = (off + j * 16 + iota) & (RING - 1)
            plsc.store_scatter(ring_rel, [pos], i_trash)
            return 0
        lax.fori_loop(0, FB // 16, padrest, 0)
        nf = (off - fired + FB - 1) // FB
        lax.fori_loop(0, nf, fire_one, fired)
        plsc.subcore_barrier()

        def oblk(blk, _):
            r0 = pl.multiple_of(s * SPAN + blk * OB, OB)
            pltpu.sync_copy(sums_sh.at[pl.ds(r0, OB)], sums_o)
            pltpu.sync_copy(cnts_sh.at[pl.ds(r0 // 4, OB // 4)], cnts_o)

            def divrow(rr, _):
                cnt = plsc.load_gather(
                    cnts_o, [i_zero + (rr >> 2), i_zero + (rr & 3) * 4])
                cm = jnp.maximum(cnt, 1.0)
                out_stage[pl.ds(rr * C, 16)] = sums_o[rr, pl.ds(0, 16)] / cm
                out_stage[pl.ds(rr * C + 16, 16)] = \
                    sums_o[rr, pl.ds(16, 16)] / cm
                return 0

            lax.fori_loop(0, OB, divrow, 0)
            pltpu.sync_copy(
                out_stage,
                out_hbm.at[pl.ds(pl.multiple_of((base + r0) * C, 8), OB * C)])
            return 0
        lax.fori_loop(0, SPAN // OB, oblk, 0)
        plsc.subcore_barrier()


@jax.jit
def _pooled(features, coords):
    mesh = plsc.VectorSubcoreMesh(core_axis_name="c", subcore_axis_name="s")
    f = pl.kernel(
        _body,
        out_type=jax.ShapeDtypeStruct((NUM_OUT * C,), jnp.float32),
        mesh=mesh,
        compiler_params=pltpu.CompilerParams(needs_layout_passes=False,
                                             use_tc_tiling_on_sc=False),
        scratch_types=[
            pltpu.VMEM_SHARED((S + 1, C), jnp.float32),
            pltpu.VMEM_SHARED((S // 4 + 1, 16), jnp.float32),
            pltpu.VMEM((2 * CW,), jnp.int32),
            pltpu.VMEM((2 * FW,), jnp.float32),
            pltpu.VMEM((BLK + 16,), jnp.int32),
            pltpu.VMEM((BLK + 16,), jnp.int32),
            pltpu.VMEM((RING,), jnp.int32),
            pltpu.VMEM((RING, C), jnp.float32),
            pltpu.VMEM((2, FB), jnp.int32),
            pltpu.VMEM((FB, 16), jnp.float32),
            pltpu.VMEM((OB, C), jnp.float32),
            pltpu.VMEM((OB // 4, 16), jnp.float32),
            pltpu.VMEM((OB * C,), jnp.float32),
            pltpu.SemaphoreType.DMA((2,)),
            pltpu.SemaphoreType.DMA((2,)),
        ],
    )
    return f(features, coords)


def kernel(features, coords):
    fpad = NBLK * FW - N * C
    cpad = NBLK * CW - N * 3
    feats1 = jnp.pad(features.reshape(-1), (0, fpad))
    coords1 = jnp.pad(coords.reshape(-1), (0, cpad), constant_values=255)
    out = _pooled(feats1, coords1)
    return out.reshape(NUM_OUT, C)


# speedup vs baseline: 1.4682x; 1.0143x over previous
import jax
import jax.numpy as jnp
from jax import lax
from jax.experimental import pallas as pl
from jax.experimental.pallas import tpu as pltpu
from jax.experimental.pallas import tpu_sc as plsc

N = 1000000
C = 32
NUM_OUT = 262144
NUM_RANGES = 8
S = NUM_OUT // NUM_RANGES
PASSES = NUM_RANGES // 2
BLK = 256
NBLK = (N + BLK - 1) // BLK
GPB = BLK // 16
CW = BLK * 3
FW = BLK * C
RING = 512
FB = 128
OB = 64
SPAN = S // 16


def _body(feat_hbm, coords_hbm, out_hbm,
          sums_sh, cnts_sh,
          cbuf, fbuf, srcr_v, brel_v, ring_rel, cr_v,
          idx_rows, ones_st, sums_o, cnts_o, out_stage,
          csems, fsems):
    c = lax.axis_index("c")
    s = lax.axis_index("s")
    iota = lax.iota(jnp.int32, 16)
    f_one = jnp.full((16,), 1.0, jnp.float32)
    f_zero = jnp.full((16,), 0.0, jnp.float32)
    i_zero = jnp.full((16,), 0, jnp.int32)
    i_trash = jnp.full((16,), S, jnp.int32)

    def init_z(i, _):
        ones_st[i] = f_zero
        return 0
    lax.fori_loop(0, FB, init_z, 0)

    n_my = jnp.where(s < NBLK - 16 * (NBLK // 16), NBLK // 16 + 1,
                     NBLK // 16)

    def start_block(k, slot):
        b = s + k * 16
        fo = jnp.minimum(b * FW, N * C - FW)
        pltpu.async_copy(coords_hbm.at[pl.ds(b * CW, CW)],
                         cbuf.at[pl.ds(slot * CW, CW)], csems.at[slot])
        pltpu.async_copy(feat_hbm.at[pl.ds(fo, FW)],
                         fbuf.at[pl.ds(slot * FW, FW)], fsems.at[slot])

    def wait_block(k, slot):
        b = s + k * 16
        fo = jnp.minimum(b * FW, N * C - FW)
        pltpu.make_async_copy(coords_hbm.at[pl.ds(b * CW, CW)],
                              cbuf.at[pl.ds(slot * CW, CW)],
                              csems.at[slot]).wait()
        pltpu.make_async_copy(feat_hbm.at[pl.ds(fo, FW)],
                              fbuf.at[pl.ds(slot * FW, FW)],
                              fsems.at[slot]).wait()

    for p in range(PASSES):
        rng = c * PASSES + p
        base = rng * S

        def zrow(i, _):
            sums_o[i, pl.ds(0, 16)] = f_zero
            sums_o[i, pl.ds(16, 16)] = f_zero
            return 0
        lax.fori_loop(0, OB, zrow, 0)

        def zcrow(i, _):
            cnts_o[i] = f_zero
            return 0
        lax.fori_loop(0, OB // 4, zcrow, 0)

        def zblk(blk, _):
            r0 = pl.multiple_of(s * SPAN + blk * OB, OB)
            pltpu.sync_copy(sums_o, sums_sh.at[pl.ds(r0, OB)])
            pltpu.sync_copy(cnts_o, cnts_sh.at[pl.ds(r0 // 4, OB // 4)])
            return 0
        lax.fori_loop(0, SPAN // OB, zblk, 0)
        plsc.subcore_barrier()

        def fire_one(i, f0):
            h0 = pl.multiple_of((f0 + i * FB) & (RING - 1), FB)
            for t in range(FB // 16):
                rl = ring_rel[pl.ds(h0 + t * 16, 16)]
                idx_rows[0, pl.ds(t * 16, 16)] = rl
                idx_rows[1, pl.ds(t * 16, 16)] = rl >> 2
                plsc.store_scatter(ones_st, [t * 16 + iota, (rl & 3) * 4],
                                   f_one)
            pltpu.sync_copy(cr_v.at[pl.ds(h0, FB)],
                            sums_sh.at[idx_rows.at[0]], add=True)
            pltpu.sync_copy(ones_st, cnts_sh.at[idx_rows.at[1]], add=True)
            for t in range(FB // 16):
                rl = idx_rows[0, pl.ds(t * 16, 16)]
                plsc.store_scatter(ones_st, [t * 16 + iota, (rl & 3) * 4],
                                   f_zero)
            return f0

        def do_block(k, carry):
            off, fired = carry
            slot = k & 1

            @pl.when(k + 1 < n_my)
            def _():
                start_block(k + 1, slot ^ 1)

            wait_block(k, slot)
            c0 = slot * CW
            b = s + k * 16
            shift = b * BLK - jnp.minimum(b * FW, N * C - FW) // C
            f0w = slot * FW + shift * C

            def scan16(g, bs):
                w = c0 + g * 48 + iota * 3
                x = plsc.load_gather(cbuf, [w])
                y = plsc.load_gather(cbuf, [w + 1])
                z = plsc.load_gather(cbuf, [w + 2])
                seg = ((x >> 1) << 12) | ((y >> 1) << 6) | (z >> 1)
                m = (seg >> 15) == rng
                rel = seg & (S - 1)
                plsc.store_compressed(srcr_v.at[pl.ds(bs, 16)], g * 16 + iota,
                                      mask=m)
                plsc.store_compressed(brel_v.at[pl.ds(bs, 16)], rel, mask=m)
                return bs + jnp.sum(m.astype(jnp.int32))

            bs = lax.fori_loop(0, GPB, scan16, 0)

            srcr_v[pl.ds(bs, 16)] = i_zero
            brel_v[pl.ds(bs, 16)] = i_trash

            def compact(gi, _):
                sr = srcr_v[pl.ds(gi * 16, 16)]
                rl = brel_v[pl.ds(gi * 16, 16)]
                pos = (off + gi * 16 + iota) & (RING - 1)
                plsc.store_scatter(ring_rel, [pos], rl)

                def xch(ch, _):
                    v = plsc.load_gather(fbuf, [f0w + sr * C + ch])
                    plsc.store_scatter(cr_v, [pos, i_zero + ch], v)
                    return 0
                lax.fori_loop(0, C, xch, 0)
                return 0

            lax.fori_loop(0, (bs + 15) // 16, compact, 0)
            off = off + bs

            nf = (off - fired) // FB
            lax.fori_loop(0, nf, fire_one, fired)
            return (off, fired + nf * FB)

        start_block(0, 0)
        off, fired = lax.fori_loop(0, n_my, do_block, (0, 0))

        def padrest(j, _):
            pos = (off + j * 16 + iota) & (RING - 1)
            plsc.store_scatter(ring_rel, [pos], i_trash)
            return 0
        lax.fori_loop(0, FB // 16, padrest, 0)
        nf = (off - fired + FB - 1) // FB
        lax.fori_loop(0, nf, fire_one, fired)
        plsc.subcore_barrier()

        def oblk(blk, _):
            r0 = pl.multiple_of(s * SPAN + blk * OB, OB)
            pltpu.sync_copy(sums_sh.at[pl.ds(r0, OB)], sums_o)
            pltpu.sync_copy(cnts_sh.at[pl.ds(r0 // 4, OB // 4)], cnts_o)

            def divrow(rr, _):
                cnt = plsc.load_gather(
                    cnts_o, [i_zero + (rr >> 2), i_zero + (rr & 3) * 4])
                cm = jnp.maximum(cnt, 1.0)
                out_stage[pl.ds(rr * C, 16)] = sums_o[rr, pl.ds(0, 16)] / cm
                out_stage[pl.ds(rr * C + 16, 16)] = \
                    sums_o[rr, pl.ds(16, 16)] / cm
                return 0

            lax.fori_loop(0, OB, divrow, 0)
            pltpu.sync_copy(
                out_stage,
                out_hbm.at[pl.ds(pl.multiple_of((base + r0) * C, 8), OB * C)])
            return 0
        lax.fori_loop(0, SPAN // OB, oblk, 0)
        plsc.subcore_barrier()


@jax.jit
def _pooled(features, coords):
    mesh = plsc.VectorSubcoreMesh(core_axis_name="c", subcore_axis_name="s")
    f = pl.kernel(
        _body,
        out_type=jax.ShapeDtypeStruct((NUM_OUT * C,), jnp.float32),
        mesh=mesh,
        compiler_params=pltpu.CompilerParams(needs_layout_passes=False,
                                             use_tc_tiling_on_sc=False),
        scratch_types=[
            pltpu.VMEM_SHARED((S + 1, C), jnp.float32),
            pltpu.VMEM_SHARED((S // 4 + 1, 16), jnp.float32),
            pltpu.VMEM((2 * CW,), jnp.int32),
            pltpu.VMEM((2 * FW,), jnp.float32),
            pltpu.VMEM((BLK + 16,), jnp.int32),
            pltpu.VMEM((BLK + 16,), jnp.int32),
            pltpu.VMEM((RING,), jnp.int32),
            pltpu.VMEM((RING, C), jnp.float32),
            pltpu.VMEM((2, FB), jnp.int32),
            pltpu.VMEM((FB, 16), jnp.float32),
            pltpu.VMEM((OB, C), jnp.float32),
            pltpu.VMEM((OB // 4, 16), jnp.float32),
            pltpu.VMEM((OB * C,), jnp.float32),
            pltpu.SemaphoreType.DMA((2,)),
            pltpu.SemaphoreType.DMA((2,)),
        ],
    )
    return f(features, coords)


def kernel(features, coords):
    cpad = NBLK * CW - N * 3
    coords1 = jnp.pad(coords.reshape(-1), (0, cpad), constant_values=255)
    out = _pooled(features.reshape(-1), coords1)
    return out.reshape(NUM_OUT, C)
